# baseline (speedup 1.0000x reference)
import functools

import jax
import jax.numpy as jnp
from jax import lax
from jax.experimental import pallas as pl
from jax.experimental.pallas import tpu as pltpu
from jax.experimental.pallas import tpu_sc as plsc

ROWS = 1024
SEGS = 50
VOCAB = 1000
NC, NS, L = 2, 16, 16
NW = NC * NS
ROWS_PER_W = ROWS // NW

_SEG_STARTS = (0, 16, 32, 34)
_ZERO_STARTS = tuple(range(0, VOCAB - L + 1, L)) + (VOCAB - L,)


def _onehot_body(x_hbm, out_hbm, buf0, buf1, idx_v, sem0, sem1):
    wid = lax.axis_index("s") * NC + lax.axis_index("c")
    base = wid * ROWS_PER_W
    bufs = (buf0, buf1)
    sems = (sem0, sem1)

    iota = lax.iota(jnp.int32, L)
    zeros = jnp.zeros((L,), jnp.float32)
    ones = jnp.ones((L,), jnp.float32)

    pltpu.sync_copy(x_hbm.at[pl.ds(base, ROWS_PER_W), :], idx_v)

    def zero_row(r, _):
        for s in _ZERO_STARTS:
            buf0[r, pl.ds(s, L)] = zeros
            buf1[r, pl.ds(s, L)] = zeros
        return 0

    lax.fori_loop(0, SEGS, zero_row, 0)

    def scatter(buf, c, vals):
        for s in _SEG_STARTS:
            cols = idx_v[c, pl.ds(s, L)]
            plsc.store_scatter(buf, [s + iota, cols], vals)

    def do_pair(g, _):
        for b in range(2):
            c = 2 * g + b

            @pl.when(g > 0)
            def _():
                pltpu.make_async_copy(
                    bufs[b], out_hbm.at[base + c - 2], sems[b]
                ).wait()
                scatter(bufs[b], c - 2, zeros)

            scatter(bufs[b], c, ones)
            pltpu.async_copy(bufs[b], out_hbm.at[base + c], sems[b])
        return 0

    lax.fori_loop(0, ROWS_PER_W // 2, do_pair, 0)

    for b in range(2):
        c = ROWS_PER_W - 2 + b
        pltpu.make_async_copy(bufs[b], out_hbm.at[base + c], sems[b]).wait()


@functools.partial(jax.jit, static_argnums=())
def _onehot_sc(x):
    mesh = plsc.VectorSubcoreMesh(core_axis_name="c", subcore_axis_name="s")
    return pl.kernel(
        _onehot_body,
        out_type=jax.ShapeDtypeStruct((ROWS, SEGS, VOCAB), jnp.float32),
        mesh=mesh,
        scratch_types=[
            pltpu.VMEM((SEGS, VOCAB), jnp.float32),
            pltpu.VMEM((SEGS, VOCAB), jnp.float32),
            pltpu.VMEM((ROWS_PER_W, SEGS), jnp.int32),
            pltpu.SemaphoreType.DMA,
            pltpu.SemaphoreType.DMA,
        ],
        compiler_params=pltpu.CompilerParams(needs_layout_passes=False),
    )(x)


def kernel(x):
    return _onehot_sc(x.astype(jnp.int32))


# speedup vs baseline: 2.5109x; 2.5109x over previous
import functools

import jax
import jax.numpy as jnp
from jax import lax
from jax.experimental import pallas as pl
from jax.experimental.pallas import tpu as pltpu
from jax.experimental.pallas import tpu_sc as plsc

ROWS = 1024
SEGS = 50
VOCAB = 1000
NC, NS, L = 2, 16, 16
NW = NC * NS
VB = 40
BPP = VOCAB // VB
NBLK = SEGS * BPP
KMAX = -(-NBLK // NW)
RCH = ROWS // L
RUNROLL = 8


def _onehot_body(xt_hbm, out_hbm, buf0, buf1, idx0, idx1, sem0, sem1):
    wid = lax.axis_index("s") * NC + lax.axis_index("c")
    bufs = (buf0, buf1)
    idxs = (idx0, idx1)
    sems = (sem0, sem1)

    iota = lax.iota(jnp.int32, L)
    zeros = jnp.zeros((L,), jnp.float32)
    ones = jnp.ones((L,), jnp.float32)

    def zero_row(r, _):
        for c in range(0, ROWS, L):
            buf0[r, pl.ds(c, L)] = zeros
            buf1[r, pl.ds(c, L)] = zeros
        return 0

    lax.fori_loop(0, VB, zero_row, 0)

    def scatter(buf, idx_v, v0, vals):
        def chunk_group(g, _):
            for u in range(RUNROLL):
                j = g * RUNROLL + u
                col = idx_v[pl.ds(j * L, L)]
                vl = col - v0
                mask = (vl >= 0) & (vl < VB)
                plsc.store_scatter(buf, [vl, j * L + iota], vals, mask=mask)
            return 0

        lax.fori_loop(0, RCH // RUNROLL, chunk_group, 0)

    def pair(k2, _):
        for b in range(2):
            k = 2 * k2 + b
            t = k * NW + wid
            tp = t - 2 * NW
            s = t // BPP
            v0 = (t % BPP) * VB
            sp = tp // BPP
            v0p = (tp % BPP) * VB

            @pl.when(k2 > 0)
            def _():
                pltpu.make_async_copy(
                    bufs[b], out_hbm.at[sp, pl.ds(v0p, VB), :], sems[b]
                ).wait()
                scatter(bufs[b], idxs[b], v0p, zeros)

            @pl.when(t < NBLK)
            def _():
                pltpu.sync_copy(xt_hbm.at[s], idxs[b])
                scatter(bufs[b], idxs[b], v0, ones)
                pltpu.async_copy(bufs[b], out_hbm.at[s, pl.ds(v0, VB), :], sems[b])

        return 0

    lax.fori_loop(0, KMAX // 2, pair, 0)

    for b in range(2):
        t = (KMAX - 2 + b) * NW + wid
        s = t // BPP
        v0 = (t % BPP) * VB

        @pl.when(t < NBLK)
        def _():
            pltpu.make_async_copy(
                bufs[b], out_hbm.at[s, pl.ds(v0, VB), :], sems[b]
            ).wait()


@jax.jit
def _onehot_sc(xt):
    mesh = plsc.VectorSubcoreMesh(core_axis_name="c", subcore_axis_name="s")
    return pl.kernel(
        _onehot_body,
        out_type=jax.ShapeDtypeStruct((SEGS, VOCAB, ROWS), jnp.float32),
        mesh=mesh,
        scratch_types=[
            pltpu.VMEM((VB, ROWS), jnp.float32),
            pltpu.VMEM((VB, ROWS), jnp.float32),
            pltpu.VMEM((ROWS,), jnp.int32),
            pltpu.VMEM((ROWS,), jnp.int32),
            pltpu.SemaphoreType.DMA,
            pltpu.SemaphoreType.DMA,
        ],
        compiler_params=pltpu.CompilerParams(needs_layout_passes=False),
    )(xt)


def kernel(x):
    xt = jnp.transpose(x.astype(jnp.int32))
    out_t = _onehot_sc(xt)
    return jnp.transpose(out_t, (2, 0, 1))
